# initial kernel scaffold (unmeasured)
import jax
import jax.numpy as jnp
from jax import lax
from jax.experimental import pallas as pl
from jax.experimental.pallas import tpu as pltpu

N_DEV = 4


def kernel(x, w_mat, scale_x, scale_w):
    m_per, k = x.shape
    _, n_per = w_mat.shape

    def body(x_ref, w_ref, sx_ref, sw_ref, out_ref, comm_ref,
             send_sems, recv_sems):
        my = lax.axis_index("i")
        left = (my - 1) % N_DEV
        right = (my + 1) % N_DEV

        barrier_sem = pltpu.get_barrier_semaphore()
        for nbr in (left, right):
            pl.semaphore_signal(
                barrier_sem, inc=1,
                device_id=(nbr,), device_id_type=pl.DeviceIdType.MESH,
            )
        pl.semaphore_wait(barrier_sem, 2)

        scale = sx_ref[0] * sw_ref[0]

        def do_chunk(src, origin):
            acc = jnp.dot(src, w_ref[...], preferred_element_type=jnp.float32)
            out_ref[pl.ds(origin * m_per, m_per), :] = jnp.maximum(
                acc * scale, 0.0)

        comm_ref[0, :, :] = x_ref[:, :]

        for h in range(N_DEV - 1):
            rdma = pltpu.make_async_remote_copy(
                src_ref=comm_ref.at[h],
                dst_ref=comm_ref.at[h + 1],
                send_sem=send_sems.at[h],
                recv_sem=recv_sems.at[h],
                device_id=(right,),
                device_id_type=pl.DeviceIdType.MESH,
            )
            rdma.start()
            do_chunk(comm_ref[h, :, :], (my - h) % N_DEV)
            rdma.wait()

        do_chunk(comm_ref[N_DEV - 1, :, :], (my - (N_DEV - 1)) % N_DEV)

    return pl.pallas_call(
        body,
        out_shape=jax.ShapeDtypeStruct((N_DEV * m_per, n_per), jnp.float32),
        in_specs=[
            pl.BlockSpec(memory_space=pltpu.VMEM),
            pl.BlockSpec(memory_space=pltpu.VMEM),
            pl.BlockSpec(memory_space=pltpu.SMEM),
            pl.BlockSpec(memory_space=pltpu.SMEM),
        ],
        out_specs=pl.BlockSpec(memory_space=pltpu.VMEM),
        scratch_shapes=[
            pltpu.VMEM((N_DEV, m_per, k), x.dtype),
            pltpu.SemaphoreType.DMA((N_DEV - 1,)),
            pltpu.SemaphoreType.DMA((N_DEV - 1,)),
        ],
        compiler_params=pltpu.CompilerParams(collective_id=0),
    )(x, w_mat, scale_x, scale_w)


# baseline (device time: 197306 ns/iter reference)
import jax
import jax.numpy as jnp
from jax import lax
from jax.experimental import pallas as pl
from jax.experimental.pallas import tpu as pltpu

N_DEV = 4


def kernel(x, w_mat, scale_x, scale_w):
    m_per, k = x.shape
    _, n_per = w_mat.shape

    x8 = x.astype(jnp.float8_e4m3fn)
    w8 = w_mat.astype(jnp.float8_e4m3fn)

    def body(x_ref, w_ref, sx_ref, sw_ref, out_hbm, comm_ref, stage_ref,
             send_sems, recv_sems, out_sems):
        my = lax.axis_index("i")
        left = (my - 1) % N_DEV
        right = (my + 1) % N_DEV

        barrier_sem = pltpu.get_barrier_semaphore()
        for nbr in (left, right):
            pl.semaphore_signal(
                barrier_sem, inc=1,
                device_id=(nbr,), device_id_type=pl.DeviceIdType.MESH,
            )
        pl.semaphore_wait(barrier_sem, 2)

        scale = sx_ref[0] * sw_ref[0]
        pending = [None, None]

        def do_chunk(i, src, origin):
            s = i % 2
            if pending[s] is not None:
                pending[s].wait()
            acc = jnp.dot(src, w_ref[...], preferred_element_type=jnp.float32)
            stage_ref[s, :, :] = jnp.maximum(acc * scale, 0.0)
            cp = pltpu.make_async_copy(
                stage_ref.at[s],
                out_hbm.at[pl.ds(origin * m_per, m_per), :],
                out_sems.at[s],
            )
            cp.start()
            pending[s] = cp

        for h in range(N_DEV - 1):
            src = x_ref if h == 0 else comm_ref.at[h - 1]
            rdma = pltpu.make_async_remote_copy(
                src_ref=src,
                dst_ref=comm_ref.at[h],
                send_sem=send_sems.at[h],
                recv_sem=recv_sems.at[h],
                device_id=(right,),
                device_id_type=pl.DeviceIdType.MESH,
            )
            rdma.start()
            held = x_ref[:, :] if h == 0 else comm_ref[h - 1, :, :]
            do_chunk(h, held, (my - h) % N_DEV)
            rdma.wait()

        do_chunk(N_DEV - 1, comm_ref[N_DEV - 2, :, :],
                 (my - (N_DEV - 1)) % N_DEV)
        pending[0].wait()
        pending[1].wait()

    return pl.pallas_call(
        body,
        out_shape=jax.ShapeDtypeStruct((N_DEV * m_per, n_per), jnp.float32),
        in_specs=[
            pl.BlockSpec(memory_space=pltpu.VMEM),
            pl.BlockSpec(memory_space=pltpu.VMEM),
            pl.BlockSpec(memory_space=pltpu.SMEM),
            pl.BlockSpec(memory_space=pltpu.SMEM),
        ],
        out_specs=pl.BlockSpec(memory_space=pl.ANY),
        scratch_shapes=[
            pltpu.VMEM((N_DEV - 1, m_per, k), jnp.float8_e4m3fn),
            pltpu.VMEM((2, m_per, n_per), jnp.float32),
            pltpu.SemaphoreType.DMA((N_DEV - 1,)),
            pltpu.SemaphoreType.DMA((N_DEV - 1,)),
            pltpu.SemaphoreType.DMA((2,)),
        ],
        compiler_params=pltpu.CompilerParams(collective_id=0),
    )(x8, w8, scale_x, scale_w)


# device time: 127170 ns/iter; 1.5515x vs baseline; 1.5515x over previous
import jax
import jax.numpy as jnp
from jax import lax
from jax.experimental import pallas as pl
from jax.experimental.pallas import tpu as pltpu

N_DEV = 4


def kernel(x, w_mat, scale_x, scale_w):
    m_per, k = x.shape
    _, n_per = w_mat.shape
    m_half = m_per // 2

    x8 = x.astype(jnp.float8_e4m3fn)
    w8 = w_mat.astype(jnp.float8_e4m3fn)

    def body(x_ref, w_ref, sx_ref, sw_ref, out_hbm,
             buf_r0, buf_r1, buf_l0, buf_l1, stage_ref,
             send_sems, recv_sems, out_sems):
        my = lax.axis_index("i")
        left = (my - 1) % N_DEV
        right = (my + 1) % N_DEV

        barrier_sem = pltpu.get_barrier_semaphore()
        for nbr in (left, right):
            pl.semaphore_signal(
                barrier_sem, inc=1,
                device_id=(nbr,), device_id_type=pl.DeviceIdType.MESH,
            )
        pl.semaphore_wait(barrier_sem, 2)

        scale = sx_ref[0] * sw_ref[0]
        pending = [None, None]
        slot = [0]

        def do_chunk(src, out_row_start, rows):
            s = slot[0] % 2
            slot[0] += 1
            if pending[s] is not None:
                pending[s].wait()
            acc = jnp.dot(src, w_ref[...], preferred_element_type=jnp.float32)
            stage_ref[s, pl.ds(0, rows), :] = jnp.maximum(acc * scale, 0.0)
            cp = pltpu.make_async_copy(
                stage_ref.at[s, pl.ds(0, rows), :],
                out_hbm.at[pl.ds(out_row_start, rows), :],
                out_sems.at[s],
            )
            cp.start()
            pending[s] = cp

        def remote(src, dst, sem_idx, target):
            return pltpu.make_async_remote_copy(
                src_ref=src, dst_ref=dst,
                send_sem=send_sems.at[sem_idx],
                recv_sem=recv_sems.at[sem_idx],
                device_id=(target,),
                device_id_type=pl.DeviceIdType.MESH,
            )

        send_r0 = remote(x_ref, buf_r0, 0, right)
        send_l0 = remote(x_ref, buf_l0, 1, left)
        send_r0.start()
        send_l0.start()

        do_chunk(x_ref[:, :], my * m_per, m_per)

        send_r0.wait_recv()
        send_r1 = remote(buf_r0.at[pl.ds(0, m_half)], buf_r1, 2, right)
        send_r1.start()
        send_l0.wait_recv()
        send_l1 = remote(buf_l0.at[pl.ds(m_half, m_half)], buf_l1, 3, left)
        send_l1.start()

        do_chunk(buf_r0[:, :], left * m_per, m_per)
        do_chunk(buf_l0[:, :], right * m_per, m_per)

        opp = (my + 2) % N_DEV
        send_r1.wait_recv()
        do_chunk(buf_r1[:, :], opp * m_per, m_half)
        send_l1.wait_recv()
        do_chunk(buf_l1[:, :], opp * m_per + m_half, m_half)

        for s in (send_r0, send_l0, send_r1, send_l1):
            s.wait_send()
        pending[0].wait()
        pending[1].wait()

    return pl.pallas_call(
        body,
        out_shape=jax.ShapeDtypeStruct((N_DEV * m_per, n_per), jnp.float32),
        in_specs=[
            pl.BlockSpec(memory_space=pltpu.VMEM),
            pl.BlockSpec(memory_space=pltpu.VMEM),
            pl.BlockSpec(memory_space=pltpu.SMEM),
            pl.BlockSpec(memory_space=pltpu.SMEM),
        ],
        out_specs=pl.BlockSpec(memory_space=pl.ANY),
        scratch_shapes=[
            pltpu.VMEM((m_per, k), jnp.float8_e4m3fn),
            pltpu.VMEM((m_half, k), jnp.float8_e4m3fn),
            pltpu.VMEM((m_per, k), jnp.float8_e4m3fn),
            pltpu.VMEM((m_half, k), jnp.float8_e4m3fn),
            pltpu.VMEM((2, m_per, n_per), jnp.float32),
            pltpu.SemaphoreType.DMA((4,)),
            pltpu.SemaphoreType.DMA((4,)),
            pltpu.SemaphoreType.DMA((2,)),
        ],
        compiler_params=pltpu.CompilerParams(collective_id=0),
    )(x8, w8, scale_x, scale_w)


# device time: 123852 ns/iter; 1.5931x vs baseline; 1.0268x over previous
import jax
import jax.numpy as jnp
from jax import lax
from jax.experimental import pallas as pl
from jax.experimental.pallas import tpu as pltpu

N_DEV = 4


def kernel(x, w_mat, scale_x, scale_w):
    m_per, k = x.shape
    _, n_per = w_mat.shape
    m_half = m_per // 2
    m_q = m_per // 4

    x8 = x.astype(jnp.float8_e4m3fn)
    w8 = w_mat.astype(jnp.float8_e4m3fn)

    def body(x_ref, w_ref, sx_ref, sw_ref, out_hbm,
             buf_r0, buf_r1, buf_l0, buf_l1, stage_ref,
             send_sems, recv_sems, out_sems):
        my = lax.axis_index("i")
        left = (my - 1) % N_DEV
        right = (my + 1) % N_DEV

        barrier_sem = pltpu.get_barrier_semaphore()
        for nbr in (left, right):
            pl.semaphore_signal(
                barrier_sem, inc=1,
                device_id=(nbr,), device_id_type=pl.DeviceIdType.MESH,
            )
        pl.semaphore_wait(barrier_sem, 2)

        scale = sx_ref[0] * sw_ref[0]
        pending = [None, None]
        slot = [0]

        def do_rows(src, src_row, out_row_start, rows):
            s = slot[0] % 2
            slot[0] += 1
            if pending[s] is not None:
                pending[s].wait()
            acc = jnp.dot(src[pl.ds(src_row, rows), :], w_ref[...],
                          preferred_element_type=jnp.float32)
            stage_ref[s, pl.ds(0, rows), :] = jnp.maximum(acc * scale, 0.0)
            cp = pltpu.make_async_copy(
                stage_ref.at[s, pl.ds(0, rows), :],
                out_hbm.at[pl.ds(out_row_start, rows), :],
                out_sems.at[s],
            )
            cp.start()
            pending[s] = cp

        def do_chunk(src, origin_row_start):
            for r in range(0, m_per, m_q):
                do_rows(src, r, origin_row_start + r, m_q)

        def remote(src, dst, sem_idx, target):
            return pltpu.make_async_remote_copy(
                src_ref=src, dst_ref=dst,
                send_sem=send_sems.at[sem_idx],
                recv_sem=recv_sems.at[sem_idx],
                device_id=(target,),
                device_id_type=pl.DeviceIdType.MESH,
            )

        send_r0 = remote(x_ref, buf_r0, 0, right)
        send_l0 = remote(x_ref, buf_l0, 1, left)
        send_r0.start()
        send_l0.start()

        do_chunk(x_ref, my * m_per)

        send_r0.wait_recv()
        fwd_r = [
            remote(buf_r0.at[pl.ds(q * m_q, m_q)], buf_r1.at[q], 2 + q, right)
            for q in range(2)
        ]
        for f in fwd_r:
            f.start()
        send_l0.wait_recv()
        fwd_l = [
            remote(buf_l0.at[pl.ds(m_half + q * m_q, m_q)], buf_l1.at[q],
                   4 + q, left)
            for q in range(2)
        ]
        for f in fwd_l:
            f.start()

        do_chunk(buf_r0, left * m_per)
        do_chunk(buf_l0, right * m_per)

        opp = (my + 2) % N_DEV
        for q in range(2):
            fwd_r[q].wait_recv()
            do_rows(buf_r1.at[q], 0, opp * m_per + q * m_q, m_q)
        for q in range(2):
            fwd_l[q].wait_recv()
            do_rows(buf_l1.at[q], 0, opp * m_per + m_half + q * m_q, m_q)

        for s in (send_r0, send_l0, *fwd_r, *fwd_l):
            s.wait_send()
        pending[0].wait()
        pending[1].wait()

    return pl.pallas_call(
        body,
        out_shape=jax.ShapeDtypeStruct((N_DEV * m_per, n_per), jnp.float32),
        in_specs=[
            pl.BlockSpec(memory_space=pltpu.VMEM),
            pl.BlockSpec(memory_space=pltpu.VMEM),
            pl.BlockSpec(memory_space=pltpu.SMEM),
            pl.BlockSpec(memory_space=pltpu.SMEM),
        ],
        out_specs=pl.BlockSpec(memory_space=pl.ANY),
        scratch_shapes=[
            pltpu.VMEM((m_per, k), jnp.float8_e4m3fn),
            pltpu.VMEM((2, m_q, k), jnp.float8_e4m3fn),
            pltpu.VMEM((m_per, k), jnp.float8_e4m3fn),
            pltpu.VMEM((2, m_q, k), jnp.float8_e4m3fn),
            pltpu.VMEM((2, m_q, n_per), jnp.float32),
            pltpu.SemaphoreType.DMA((6,)),
            pltpu.SemaphoreType.DMA((6,)),
            pltpu.SemaphoreType.DMA((2,)),
        ],
        compiler_params=pltpu.CompilerParams(collective_id=0),
    )(x8, w8, scale_x, scale_w)


# device time: 120225 ns/iter; 1.6411x vs baseline; 1.0302x over previous
import jax
import jax.numpy as jnp
from jax import lax
from jax.experimental import pallas as pl
from jax.experimental.pallas import tpu as pltpu

N_DEV = 4


def kernel(x, w_mat, scale_x, scale_w):
    m_per, k = x.shape
    _, n_per = w_mat.shape
    m_half = m_per // 2
    m_q = m_per // 4

    x8 = x.astype(jnp.float8_e4m3fn)
    w8 = w_mat.astype(jnp.float8_e4m3fn)

    def body(x_ref, w_ref, sx_ref, sw_ref, out_hbm,
             buf_r0, buf_r1, buf_l0, buf_l1, stage_ref,
             send_sems, recv_sems, out_sems):
        my = lax.axis_index("i")
        left = (my - 1) % N_DEV
        right = (my + 1) % N_DEV

        barrier_sem = pltpu.get_barrier_semaphore()
        for nbr in (left, right):
            pl.semaphore_signal(
                barrier_sem, inc=1,
                device_id=(nbr,), device_id_type=pl.DeviceIdType.MESH,
            )
        pl.semaphore_wait(barrier_sem, 2)

        scale = sx_ref[0] * sw_ref[0]
        pending = [None, None]
        slot = [0]

        def do_rows(src, src_row, out_row_start, rows):
            s = slot[0] % 2
            slot[0] += 1
            if pending[s] is not None:
                pending[s].wait()
            acc = jnp.dot(src[pl.ds(src_row, rows), :], w_ref[...],
                          preferred_element_type=jnp.float32)
            stage_ref[s, pl.ds(0, rows), :] = jnp.maximum(acc * scale, 0.0)
            cp = pltpu.make_async_copy(
                stage_ref.at[s, pl.ds(0, rows), :],
                out_hbm.at[pl.ds(out_row_start, rows), :],
                out_sems.at[s],
            )
            cp.start()
            pending[s] = cp

        def do_chunk(src, origin_row_start):
            for r in range(0, m_per, m_q):
                do_rows(src, r, origin_row_start + r, m_q)

        def remote(src, dst, sem_idx, target):
            return pltpu.make_async_remote_copy(
                src_ref=src, dst_ref=dst,
                send_sem=send_sems.at[sem_idx],
                recv_sem=recv_sems.at[sem_idx],
                device_id=(target,),
                device_id_type=pl.DeviceIdType.MESH,
            )

        hop1_r = [
            remote(x_ref.at[pl.ds(q * m_q, m_q)],
                   buf_r0.at[pl.ds(q * m_q, m_q)], q, right)
            for q in range(4)
        ]
        hop1_l = [
            remote(x_ref.at[pl.ds(q * m_q, m_q)],
                   buf_l0.at[pl.ds(q * m_q, m_q)], 4 + q, left)
            for q in range(4)
        ]
        for f in hop1_r + hop1_l:
            f.start()

        do_chunk(x_ref, my * m_per)

        fwd_r = [None, None]
        fwd_l = [None, None]
        for q in range(4):
            hop1_r[q].wait_recv()
            if q < 2:
                fwd_r[q] = remote(buf_r0.at[pl.ds(q * m_q, m_q)],
                                  buf_r1.at[q], 8 + q, right)
                fwd_r[q].start()
            do_rows(buf_r0, q * m_q, left * m_per + q * m_q, m_q)
            hop1_l[q].wait_recv()
            if q >= 2:
                fwd_l[q - 2] = remote(buf_l0.at[pl.ds(q * m_q, m_q)],
                                      buf_l1.at[q - 2], 8 + q, left)
                fwd_l[q - 2].start()
            do_rows(buf_l0, q * m_q, right * m_per + q * m_q, m_q)

        opp = (my + 2) % N_DEV
        for q in range(2):
            fwd_r[q].wait_recv()
            do_rows(buf_r1.at[q], 0, opp * m_per + q * m_q, m_q)
            fwd_l[q].wait_recv()
            do_rows(buf_l1.at[q], 0, opp * m_per + m_half + q * m_q, m_q)

        for s in hop1_r + hop1_l + fwd_r + fwd_l:
            s.wait_send()
        pending[0].wait()
        pending[1].wait()

    return pl.pallas_call(
        body,
        out_shape=jax.ShapeDtypeStruct((N_DEV * m_per, n_per), jnp.float32),
        in_specs=[
            pl.BlockSpec(memory_space=pltpu.VMEM),
            pl.BlockSpec(memory_space=pltpu.VMEM),
            pl.BlockSpec(memory_space=pltpu.SMEM),
            pl.BlockSpec(memory_space=pltpu.SMEM),
        ],
        out_specs=pl.BlockSpec(memory_space=pl.ANY),
        scratch_shapes=[
            pltpu.VMEM((m_per, k), jnp.float8_e4m3fn),
            pltpu.VMEM((2, m_q, k), jnp.float8_e4m3fn),
            pltpu.VMEM((m_per, k), jnp.float8_e4m3fn),
            pltpu.VMEM((2, m_q, k), jnp.float8_e4m3fn),
            pltpu.VMEM((2, m_q, n_per), jnp.float32),
            pltpu.SemaphoreType.DMA((12,)),
            pltpu.SemaphoreType.DMA((12,)),
            pltpu.SemaphoreType.DMA((2,)),
        ],
        compiler_params=pltpu.CompilerParams(collective_id=0),
    )(x8, w8, scale_x, scale_w)


# device time: 108451 ns/iter; 1.8193x vs baseline; 1.1086x over previous
import jax
import jax.numpy as jnp
from jax import lax
from jax.experimental import pallas as pl
from jax.experimental.pallas import tpu as pltpu

N_DEV = 4


def kernel(x, w_mat, scale_x, scale_w):
    m_per, k = x.shape
    _, n_per = w_mat.shape
    m_half = m_per // 2
    m_q = m_per // 4

    x8 = x.astype(jnp.float8_e4m3fn)
    wblk = 256

    def body(x_ref, w_hbm, sx_ref, sw_ref, out_hbm,
             w_ref, wstage_ref, buf_r0, buf_r1, buf_l0, buf_l1, stage_ref,
             send_sems, recv_sems, out_sems, w_sems):
        my = lax.axis_index("i")
        left = (my - 1) % N_DEV
        right = (my + 1) % N_DEV

        barrier_sem = pltpu.get_barrier_semaphore()
        for nbr in (left, right):
            pl.semaphore_signal(
                barrier_sem, inc=1,
                device_id=(nbr,), device_id_type=pl.DeviceIdType.MESH,
            )
        pl.semaphore_wait(barrier_sem, 2)

        scale = sx_ref[0] * sw_ref[0]
        pending = [None, None]
        slot = [0]

        def do_rows(src, src_row, out_row_start, rows):
            s = slot[0] % 2
            slot[0] += 1
            if pending[s] is not None:
                pending[s].wait()
            acc = jnp.dot(src[pl.ds(src_row, rows), :], w_ref[...],
                          preferred_element_type=jnp.float32)
            stage_ref[s, pl.ds(0, rows), :] = jnp.maximum(acc * scale, 0.0)
            cp = pltpu.make_async_copy(
                stage_ref.at[s, pl.ds(0, rows), :],
                out_hbm.at[pl.ds(out_row_start, rows), :],
                out_sems.at[s],
            )
            cp.start()
            pending[s] = cp

        def do_chunk(src, origin_row_start):
            for r in range(0, m_per, m_q):
                do_rows(src, r, origin_row_start + r, m_q)

        def remote(src, dst, sem_idx, target):
            return pltpu.make_async_remote_copy(
                src_ref=src, dst_ref=dst,
                send_sem=send_sems.at[sem_idx],
                recv_sem=recv_sems.at[sem_idx],
                device_id=(target,),
                device_id_type=pl.DeviceIdType.MESH,
            )

        hop1_r = [
            remote(x_ref.at[pl.ds(q * m_q, m_q)],
                   buf_r0.at[pl.ds(q * m_q, m_q)], q, right)
            for q in range(4)
        ]
        hop1_l = [
            remote(x_ref.at[pl.ds(q * m_q, m_q)],
                   buf_l0.at[pl.ds(q * m_q, m_q)], 4 + q, left)
            for q in range(4)
        ]
        for f in hop1_r + hop1_l:
            f.start()

        def wdma(b, sl):
            return pltpu.make_async_copy(
                w_hbm.at[pl.ds(b * wblk, wblk), :], wstage_ref.at[sl],
                w_sems.at[sl])

        nblk = k // wblk
        wdma(0, 0).start()
        for b in range(nblk):
            if b + 1 < nblk:
                wdma(b + 1, (b + 1) % 2).start()
            wdma(b, b % 2).wait()
            w_ref[pl.ds(b * wblk, wblk), :] = (
                wstage_ref[b % 2].astype(jnp.float8_e4m3fn))

        do_chunk(x_ref, my * m_per)

        fwd_r = [None, None]
        fwd_l = [None, None]
        for q in range(4):
            hop1_r[q].wait_recv()
            if q < 2:
                fwd_r[q] = remote(buf_r0.at[pl.ds(q * m_q, m_q)],
                                  buf_r1.at[q], 8 + q, right)
                fwd_r[q].start()
            do_rows(buf_r0, q * m_q, left * m_per + q * m_q, m_q)
            hop1_l[q].wait_recv()
            if q >= 2:
                fwd_l[q - 2] = remote(buf_l0.at[pl.ds(q * m_q, m_q)],
                                      buf_l1.at[q - 2], 8 + q, left)
                fwd_l[q - 2].start()
            do_rows(buf_l0, q * m_q, right * m_per + q * m_q, m_q)

        opp = (my + 2) % N_DEV
        for q in range(2):
            fwd_r[q].wait_recv()
            do_rows(buf_r1.at[q], 0, opp * m_per + q * m_q, m_q)
            fwd_l[q].wait_recv()
            do_rows(buf_l1.at[q], 0, opp * m_per + m_half + q * m_q, m_q)

        for s in hop1_r + hop1_l + fwd_r + fwd_l:
            s.wait_send()
        pending[0].wait()
        pending[1].wait()

    return pl.pallas_call(
        body,
        out_shape=jax.ShapeDtypeStruct((N_DEV * m_per, n_per), jnp.float32),
        in_specs=[
            pl.BlockSpec(memory_space=pltpu.VMEM),
            pl.BlockSpec(memory_space=pl.ANY),
            pl.BlockSpec(memory_space=pltpu.SMEM),
            pl.BlockSpec(memory_space=pltpu.SMEM),
        ],
        out_specs=pl.BlockSpec(memory_space=pl.ANY),
        scratch_shapes=[
            pltpu.VMEM((k, n_per), jnp.float8_e4m3fn),
            pltpu.VMEM((2, wblk, n_per), jnp.float32),
            pltpu.VMEM((m_per, k), jnp.float8_e4m3fn),
            pltpu.VMEM((2, m_q, k), jnp.float8_e4m3fn),
            pltpu.VMEM((m_per, k), jnp.float8_e4m3fn),
            pltpu.VMEM((2, m_q, k), jnp.float8_e4m3fn),
            pltpu.VMEM((2, m_q, n_per), jnp.float32),
            pltpu.SemaphoreType.DMA((12,)),
            pltpu.SemaphoreType.DMA((12,)),
            pltpu.SemaphoreType.DMA((2,)),
            pltpu.SemaphoreType.DMA((2,)),
        ],
        compiler_params=pltpu.CompilerParams(collective_id=0),
    )(x8, w_mat, scale_x, scale_w)


# device time: 107632 ns/iter; 1.8332x vs baseline; 1.0076x over previous
import jax
import jax.numpy as jnp
from jax import lax
from jax.experimental import pallas as pl
from jax.experimental.pallas import tpu as pltpu

N_DEV = 4


def kernel(x, w_mat, scale_x, scale_w):
    m_per, k = x.shape
    _, n_per = w_mat.shape
    m_half = m_per // 2
    m_q = m_per // 4

    wblk = 256

    def body(x_ref, w_hbm, sx_ref, sw_ref, out_hbm,
             x8_ref, w_ref, wstage_ref, buf_r0, buf_r1, buf_l0, buf_l1,
             stage_ref, send_sems, recv_sems, out_sems, w_sems):
        my = lax.axis_index("i")
        left = (my - 1) % N_DEV
        right = (my + 1) % N_DEV

        barrier_sem = pltpu.get_barrier_semaphore()
        for nbr in (left, right):
            pl.semaphore_signal(
                barrier_sem, inc=1,
                device_id=(nbr,), device_id_type=pl.DeviceIdType.MESH,
            )
        pl.semaphore_wait(barrier_sem, 2)

        scale = sx_ref[0] * sw_ref[0]
        pending = [None, None]
        slot = [0]

        def do_rows(src, src_row, out_row_start, rows):
            s = slot[0] % 2
            slot[0] += 1
            if pending[s] is not None:
                pending[s].wait()
            acc = jnp.dot(src[pl.ds(src_row, rows), :], w_ref[...],
                          preferred_element_type=jnp.float32)
            stage_ref[s, pl.ds(0, rows), :] = jnp.maximum(acc * scale, 0.0)
            cp = pltpu.make_async_copy(
                stage_ref.at[s, pl.ds(0, rows), :],
                out_hbm.at[pl.ds(out_row_start, rows), :],
                out_sems.at[s],
            )
            cp.start()
            pending[s] = cp

        def do_chunk(src, origin_row_start):
            for r in range(0, m_per, m_q):
                do_rows(src, r, origin_row_start + r, m_q)

        def remote(src, dst, sem_idx, target):
            return pltpu.make_async_remote_copy(
                src_ref=src, dst_ref=dst,
                send_sem=send_sems.at[sem_idx],
                recv_sem=recv_sems.at[sem_idx],
                device_id=(target,),
                device_id_type=pl.DeviceIdType.MESH,
            )

        hop1_r = [
            remote(x8_ref.at[pl.ds(q * m_q, m_q)],
                   buf_r0.at[pl.ds(q * m_q, m_q)], q, right)
            for q in range(4)
        ]
        hop1_l = [
            remote(x8_ref.at[pl.ds(q * m_q, m_q)],
                   buf_l0.at[pl.ds(q * m_q, m_q)], 4 + q, left)
            for q in range(4)
        ]
        for q in range(4):
            x8_ref[pl.ds(q * m_q, m_q), :] = (
                x_ref[pl.ds(q * m_q, m_q), :].astype(jnp.float8_e4m3fn))
            hop1_r[q].start()
            hop1_l[q].start()

        def wdma(b, sl):
            return pltpu.make_async_copy(
                w_hbm.at[pl.ds(b * wblk, wblk), :], wstage_ref.at[sl],
                w_sems.at[sl])

        nblk = k // wblk
        wdma(0, 0).start()
        for b in range(nblk):
            if b + 1 < nblk:
                wdma(b + 1, (b + 1) % 2).start()
            wdma(b, b % 2).wait()
            w_ref[pl.ds(b * wblk, wblk), :] = (
                wstage_ref[b % 2].astype(jnp.float8_e4m3fn))

        do_chunk(x8_ref, my * m_per)

        fwd_r = [None, None]
        fwd_l = [None, None]
        for q in range(4):
            hop1_r[q].wait_recv()
            if q < 2:
                fwd_r[q] = remote(buf_r0.at[pl.ds(q * m_q, m_q)],
                                  buf_r1.at[q], 8 + q, right)
                fwd_r[q].start()
            do_rows(buf_r0, q * m_q, left * m_per + q * m_q, m_q)
            hop1_l[q].wait_recv()
            if q >= 2:
                fwd_l[q - 2] = remote(buf_l0.at[pl.ds(q * m_q, m_q)],
                                      buf_l1.at[q - 2], 8 + q, left)
                fwd_l[q - 2].start()
            do_rows(buf_l0, q * m_q, right * m_per + q * m_q, m_q)

        opp = (my + 2) % N_DEV
        for q in range(2):
            fwd_r[q].wait_recv()
            do_rows(buf_r1.at[q], 0, opp * m_per + q * m_q, m_q)
            fwd_l[q].wait_recv()
            do_rows(buf_l1.at[q], 0, opp * m_per + m_half + q * m_q, m_q)

        for s in hop1_r + hop1_l + fwd_r + fwd_l:
            s.wait_send()
        pending[0].wait()
        pending[1].wait()

    return pl.pallas_call(
        body,
        out_shape=jax.ShapeDtypeStruct((N_DEV * m_per, n_per), jnp.float32),
        in_specs=[
            pl.BlockSpec(memory_space=pltpu.VMEM),
            pl.BlockSpec(memory_space=pl.ANY),
            pl.BlockSpec(memory_space=pltpu.SMEM),
            pl.BlockSpec(memory_space=pltpu.SMEM),
        ],
        out_specs=pl.BlockSpec(memory_space=pl.ANY),
        scratch_shapes=[
            pltpu.VMEM((m_per, k), jnp.float8_e4m3fn),
            pltpu.VMEM((k, n_per), jnp.float8_e4m3fn),
            pltpu.VMEM((2, wblk, n_per), jnp.float32),
            pltpu.VMEM((m_per, k), jnp.float8_e4m3fn),
            pltpu.VMEM((2, m_q, k), jnp.float8_e4m3fn),
            pltpu.VMEM((m_per, k), jnp.float8_e4m3fn),
            pltpu.VMEM((2, m_q, k), jnp.float8_e4m3fn),
            pltpu.VMEM((2, m_q, n_per), jnp.float32),
            pltpu.SemaphoreType.DMA((12,)),
            pltpu.SemaphoreType.DMA((12,)),
            pltpu.SemaphoreType.DMA((2,)),
            pltpu.SemaphoreType.DMA((2,)),
        ],
        compiler_params=pltpu.CompilerParams(collective_id=0),
    )(x, w_mat, scale_x, scale_w)


# device time: 107517 ns/iter; 1.8351x vs baseline; 1.0011x over previous
import jax
import jax.numpy as jnp
from jax import lax
from jax.experimental import pallas as pl
from jax.experimental.pallas import tpu as pltpu

N_DEV = 4


def kernel(x, w_mat, scale_x, scale_w):
    m_per, k = x.shape
    _, n_per = w_mat.shape
    m_half = m_per // 2
    m_q = m_per // 4

    wblk = 256

    def body(x_ref, w_hbm, sx_ref, sw_ref, out_hbm,
             x8_ref, w_ref, wstage_ref, buf_r0, buf_r1, buf_l0, buf_l1,
             stage_ref, send_sems, recv_sems, out_sems, w_sems):
        my = lax.axis_index("i")
        left = (my - 1) % N_DEV
        right = (my + 1) % N_DEV

        barrier_sem = pltpu.get_barrier_semaphore()
        for nbr in (left, right):
            pl.semaphore_signal(
                barrier_sem, inc=1,
                device_id=(nbr,), device_id_type=pl.DeviceIdType.MESH,
            )
        pl.semaphore_wait(barrier_sem, 2)

        scale = sx_ref[0] * sw_ref[0]
        pending = [None, None]
        slot = [0]

        def do_rows(src, src_row, out_row_start, rows):
            s = slot[0] % 2
            slot[0] += 1
            if pending[s] is not None:
                pending[s].wait()
            acc = jnp.dot(src[pl.ds(src_row, rows), :], w_ref[...],
                          preferred_element_type=jnp.float32)
            stage_ref[s, pl.ds(0, rows), :] = jnp.maximum(acc * scale, 0.0)
            cp = pltpu.make_async_copy(
                stage_ref.at[s, pl.ds(0, rows), :],
                out_hbm.at[pl.ds(out_row_start, rows), :],
                out_sems.at[s],
            )
            cp.start()
            pending[s] = cp

        def do_chunk(src, origin_row_start):
            for r in range(0, m_per, m_q):
                do_rows(src, r, origin_row_start + r, m_q)

        def remote(src, dst, sem_idx, target):
            return pltpu.make_async_remote_copy(
                src_ref=src, dst_ref=dst,
                send_sem=send_sems.at[sem_idx],
                recv_sem=recv_sems.at[sem_idx],
                device_id=(target,),
                device_id_type=pl.DeviceIdType.MESH,
            )

        hop1_r = [
            remote(x8_ref.at[pl.ds(q * m_q, m_q)],
                   buf_r0.at[pl.ds(q * m_q, m_q)], q, right)
            for q in range(4)
        ]
        hop1_l = [
            remote(x8_ref.at[pl.ds(q * m_q, m_q)],
                   buf_l0.at[pl.ds(q * m_q, m_q)], 4 + q, left)
            for q in range(4)
        ]
        for q in range(4):
            x8_ref[pl.ds(q * m_q, m_q), :] = (
                x_ref[pl.ds(q * m_q, m_q), :].astype(jnp.float8_e4m3fn))
            hop1_r[q].start()
            hop1_l[q].start()

        def wdma(b, sl):
            return pltpu.make_async_copy(
                w_hbm.at[pl.ds(b * wblk, wblk), :], wstage_ref.at[sl],
                w_sems.at[sl])

        nblk = k // wblk
        wdma(0, 0).start()
        for b in range(nblk):
            if b + 1 < nblk:
                wdma(b + 1, (b + 1) % 2).start()
            wdma(b, b % 2).wait()
            w_ref[pl.ds(b * wblk, wblk), :] = (
                wstage_ref[b % 2].astype(jnp.float8_e4m3fn))

        do_chunk(x8_ref, my * m_per)

        fwd_r = [None, None]
        fwd_l = [None, None]
        for q in range(4):
            hop1_r[q].wait_recv()
            hop1_l[q].wait_recv()
            if q < 2:
                fwd_r[q] = remote(buf_r0.at[pl.ds(q * m_q, m_q)],
                                  buf_r1.at[q], 8 + q, right)
                fwd_r[q].start()
            else:
                fwd_l[q - 2] = remote(buf_l0.at[pl.ds(q * m_q, m_q)],
                                      buf_l1.at[q - 2], 8 + q, left)
                fwd_l[q - 2].start()
            do_rows(buf_r0, q * m_q, left * m_per + q * m_q, m_q)
            do_rows(buf_l0, q * m_q, right * m_per + q * m_q, m_q)

        opp = (my + 2) % N_DEV
        for q in range(2):
            fwd_r[q].wait_recv()
            do_rows(buf_r1.at[q], 0, opp * m_per + q * m_q, m_q)
            fwd_l[q].wait_recv()
            do_rows(buf_l1.at[q], 0, opp * m_per + m_half + q * m_q, m_q)

        for s in hop1_r + hop1_l + fwd_r + fwd_l:
            s.wait_send()
        pending[0].wait()
        pending[1].wait()

    return pl.pallas_call(
        body,
        out_shape=jax.ShapeDtypeStruct((N_DEV * m_per, n_per), jnp.float32),
        in_specs=[
            pl.BlockSpec(memory_space=pltpu.VMEM),
            pl.BlockSpec(memory_space=pl.ANY),
            pl.BlockSpec(memory_space=pltpu.SMEM),
            pl.BlockSpec(memory_space=pltpu.SMEM),
        ],
        out_specs=pl.BlockSpec(memory_space=pl.ANY),
        scratch_shapes=[
            pltpu.VMEM((m_per, k), jnp.float8_e4m3fn),
            pltpu.VMEM((k, n_per), jnp.float8_e4m3fn),
            pltpu.VMEM((2, wblk, n_per), jnp.float32),
            pltpu.VMEM((m_per, k), jnp.float8_e4m3fn),
            pltpu.VMEM((2, m_q, k), jnp.float8_e4m3fn),
            pltpu.VMEM((m_per, k), jnp.float8_e4m3fn),
            pltpu.VMEM((2, m_q, k), jnp.float8_e4m3fn),
            pltpu.VMEM((2, m_q, n_per), jnp.float32),
            pltpu.SemaphoreType.DMA((12,)),
            pltpu.SemaphoreType.DMA((12,)),
            pltpu.SemaphoreType.DMA((2,)),
            pltpu.SemaphoreType.DMA((2,)),
        ],
        compiler_params=pltpu.CompilerParams(collective_id=0),
    )(x, w_mat, scale_x, scale_w)
